# baseline (speedup 1.0000x reference)
import functools

import jax
import jax.numpy as jnp
from jax import lax
from jax.experimental import pallas as pl
from jax.experimental.pallas import tpu as pltpu
from jax.experimental.pallas import tpu_sc as plsc

_B = 2
_CITY = 10000
_E = 320000
_TW = 24
_F = 8
_XEM = 64
_GNNH = 64
_PRED = 12

_N = _B * _CITY
_NE = _B * _E
_NC = 2
_NS = 16
_NW = _NC * _NS
_EPW = _NE // _NW
_CH = 80
_NCHUNK = _EPW // _CH
_NPAD = 20480
_RPT = _NPAD // _NS
_RB = 1000


def _tc1_body(x_ref, wx_ref, bx_ref, w1_ref, b1_ref, x2_ref, g0_ref):
    x2 = jnp.dot(x_ref[...], wx_ref[...], preferred_element_type=jnp.float32)
    x2 = x2 + bx_ref[...]
    x2_ref[...] = x2
    g0 = jnp.dot(x2, w1_ref[...], preferred_element_type=jnp.float32)
    g0_ref[...] = g0 + b1_ref[...]


def _tc1(xf, wx, bx, w1a, b1):
    return pl.pallas_call(
        _tc1_body,
        grid=(_N // _RB,),
        in_specs=[
            pl.BlockSpec((_RB, _TW * _F), lambda i: (i, 0)),
            pl.BlockSpec((_TW * _F, _XEM), lambda i: (0, 0)),
            pl.BlockSpec((1, _XEM), lambda i: (0, 0)),
            pl.BlockSpec((_XEM, _GNNH), lambda i: (0, 0)),
            pl.BlockSpec((1, _GNNH), lambda i: (0, 0)),
        ],
        out_specs=[
            pl.BlockSpec((_RB, _XEM), lambda i: (i, 0)),
            pl.BlockSpec((_RB, _GNNH), lambda i: (i, 0)),
        ],
        out_shape=[
            jax.ShapeDtypeStruct((_N, _XEM), jnp.float32),
            jax.ShapeDtypeStruct((_N, _GNNH), jnp.float32),
        ],
    )(xf, wx, bx, w1a, b1)


def _tc2_body(x2_ref, pa_ref, pb_ref, ca_ref, cb_ref, w2a_ref, w2b_ref, b2_ref,
              w1b_ref, b1b_ref, nx_ref, g1_ref):
    cnt = ca_ref[...][:, :1] + cb_ref[...][:, :1]
    agg = (pa_ref[...] + pb_ref[...]) / jnp.maximum(cnt, 1.0)
    nx = jnp.dot(x2_ref[...], w2a_ref[...], preferred_element_type=jnp.float32)
    nx = nx + jnp.dot(agg, w2b_ref[...], preferred_element_type=jnp.float32)
    nx = jnp.maximum(nx + b2_ref[...], 0.0)
    nx_ref[...] = nx
    g1 = jnp.dot(nx, w1b_ref[...], preferred_element_type=jnp.float32)
    g1_ref[...] = g1 + b1b_ref[...]


def _tc2(x2, pa, pb, ca, cb, w2a, w2b, b2, w1b, b1b):
    return pl.pallas_call(
        _tc2_body,
        grid=(_N // _RB,),
        in_specs=[
            pl.BlockSpec((_RB, _XEM), lambda i: (i, 0)),
            pl.BlockSpec((_RB, _GNNH), lambda i: (i, 0)),
            pl.BlockSpec((_RB, _GNNH), lambda i: (i, 0)),
            pl.BlockSpec((_RB, 8), lambda i: (i, 0)),
            pl.BlockSpec((_RB, 8), lambda i: (i, 0)),
            pl.BlockSpec((_XEM, _GNNH), lambda i: (0, 0)),
            pl.BlockSpec((_GNNH, _GNNH), lambda i: (0, 0)),
            pl.BlockSpec((1, _GNNH), lambda i: (0, 0)),
            pl.BlockSpec((_GNNH, _GNNH), lambda i: (0, 0)),
            pl.BlockSpec((1, _GNNH), lambda i: (0, 0)),
        ],
        out_specs=[
            pl.BlockSpec((_RB, _GNNH), lambda i: (i, 0)),
            pl.BlockSpec((_RB, _GNNH), lambda i: (i, 0)),
        ],
        out_shape=[
            jax.ShapeDtypeStruct((_N, _GNNH), jnp.float32),
            jax.ShapeDtypeStruct((_N, _GNNH), jnp.float32),
        ],
    )(x2, pa, pb, ca, cb, w2a, w2b, b2, w1b, b1b)


def _tc3_body(nx_ref, pa_ref, pb_ref, ca_ref, cb_ref, w2c_ref, w2d_ref,
              b2_ref, wp1_ref, bp1_ref, wp2_ref, bp2_ref, out_ref):
    cnt = ca_ref[...][:, :1] + cb_ref[...][:, :1]
    agg = (pa_ref[...] + pb_ref[...]) / jnp.maximum(cnt, 1.0)
    nx2 = jnp.dot(nx_ref[...], w2c_ref[...], preferred_element_type=jnp.float32)
    nx2 = nx2 + jnp.dot(agg, w2d_ref[...], preferred_element_type=jnp.float32)
    nx2 = jnp.maximum(nx2 + b2_ref[...], 0.0)
    h = jnp.dot(nx2, wp1_ref[...], preferred_element_type=jnp.float32)
    h = jnp.maximum(h + bp1_ref[...], 0.0)
    o = jnp.dot(h, wp2_ref[...], preferred_element_type=jnp.float32)
    out_ref[...] = jnp.maximum(o + bp2_ref[...], 0.0)


def _tc3(nx, pa, pb, ca, cb, w2c, w2d, b2, wp1, bp1, wp2, bp2):
    return pl.pallas_call(
        _tc3_body,
        grid=(_N // _RB,),
        in_specs=[
            pl.BlockSpec((_RB, _GNNH), lambda i: (i, 0)),
            pl.BlockSpec((_RB, _GNNH), lambda i: (i, 0)),
            pl.BlockSpec((_RB, _GNNH), lambda i: (i, 0)),
            pl.BlockSpec((_RB, 8), lambda i: (i, 0)),
            pl.BlockSpec((_RB, 8), lambda i: (i, 0)),
            pl.BlockSpec((_GNNH, _GNNH), lambda i: (0, 0)),
            pl.BlockSpec((_GNNH, _GNNH), lambda i: (0, 0)),
            pl.BlockSpec((1, _GNNH), lambda i: (0, 0)),
            pl.BlockSpec((_GNNH, 16), lambda i: (0, 0)),
            pl.BlockSpec((1, 16), lambda i: (0, 0)),
            pl.BlockSpec((16, _PRED), lambda i: (0, 0)),
            pl.BlockSpec((1, _PRED), lambda i: (0, 0)),
        ],
        out_specs=pl.BlockSpec((_RB, _PRED), lambda i: (i, 0)),
        out_shape=jax.ShapeDtypeStruct((_N, _PRED), jnp.float32),
    )(nx, pa, pb, ca, cb, w2c, w2d, b2, wp1, bp1, wp2, bp2)


_G = 2
_NCI = 10
_S = _NCI * _CH
_NSUP = _EPW // _S
_NR = _NCI // _G


def _make_sc():
    mesh = plsc.VectorSubcoreMesh(core_axis_name="c", subcore_axis_name="s")

    @functools.partial(
        pl.kernel,
        out_type=jax.ShapeDtypeStruct((_NC, _NPAD, 64), jnp.float32),
        mesh=mesh,
        compiler_params=pltpu.CompilerParams(use_tc_tiling_on_sc=False),
        scratch_types=[
            pltpu.VMEM((2, _NCI, _CH), jnp.int32),
            pltpu.VMEM((2, _NCI, _CH), jnp.int32),
            pltpu.VMEM((2, _S), jnp.float32),
            pltpu.VMEM((_G, _CH, 64), jnp.float32),
            pltpu.VMEM((_G, _CH, 64), jnp.float32),
            pltpu.VMEM((64,), jnp.float32),
            pltpu.VMEM_SHARED((_NPAD, 64), jnp.float32),
        ] + [pltpu.SemaphoreType.DMA] * (2 * _G + 2),
    )
    def sc(g_hbm, row_hbm, col_hbm, ew_hbm, wedge_hbm, zeros_hbm, out_hbm,
           row_v, col_v, ew_v, rows_v, msg_v, wedge_v, acc_sh, *sems):
        sg = sems[:_G]
        ss = sems[_G:2 * _G]
        sup = sems[2 * _G:]
        c = lax.axis_index("c")
        s = lax.axis_index("s")
        wid = s * _NC + c
        pltpu.sync_copy(zeros_hbm.at[pl.ds(s * _RPT, _RPT)],
                        acc_sh.at[pl.ds(s * _RPT, _RPT)])
        pltpu.sync_copy(wedge_hbm, wedge_v)
        plsc.subcore_barrier()

        cbase = wid * _NCHUNK
        ebase = wid * _EPW

        def load_super(p, slot):
            pltpu.async_copy(row_hbm.at[pl.ds(cbase + p * _NCI, _NCI)],
                             row_v.at[slot], sup[slot])
            pltpu.async_copy(col_hbm.at[pl.ds(cbase + p * _NCI, _NCI)],
                             col_v.at[slot], sup[slot])
            pltpu.async_copy(ew_hbm.at[pl.ds(ebase + p * _S, _S)],
                             ew_v.at[slot], sup[slot])

        def wait_super(slot):
            pltpu.make_async_copy(row_hbm.at[pl.ds(cbase, _NCI)],
                                  row_v.at[slot], sup[slot]).wait()
            pltpu.make_async_copy(col_hbm.at[pl.ds(cbase, _NCI)],
                                  col_v.at[slot], sup[slot]).wait()
            pltpu.make_async_copy(ew_hbm.at[pl.ds(ebase, _S)],
                                  ew_v.at[slot], sup[slot]).wait()

        def fire_gather(sp, j, g):
            pltpu.async_copy(g_hbm.at[row_v.at[sp, j]], rows_v.at[g], sg[g])

        def wait_gather(sp, j, g):
            pltpu.make_async_copy(g_hbm.at[row_v.at[sp, j]],
                                  rows_v.at[g], sg[g]).wait()

        def fire_scatter(sp, j, g):
            pltpu.async_copy(msg_v.at[g], acc_sh.at[col_v.at[sp, j]],
                             ss[g], add=True)

        def wait_scatter(sp, g):
            pltpu.make_async_copy(msg_v.at[g], acc_sh.at[col_v.at[sp, 0]],
                                  ss[g]).wait()

        pltpu.sync_copy(row_hbm.at[pl.ds(cbase, _NCI)], row_v.at[0])
        pltpu.sync_copy(col_hbm.at[pl.ds(cbase, _NCI)], col_v.at[0])
        pltpu.sync_copy(ew_hbm.at[pl.ds(ebase, _S)], ew_v.at[0])

        def super_phase(q, sp, tail=False):
            p = 2 * q + sp
            if tail:
                wait_super(sp)
            elif sp == 0:
                @pl.when(q > 0)
                def _():
                    wait_super(0)
                load_super(p + 1, 1)
            else:
                wait_super(1)
                load_super(p + 1, 0)
            for g in range(_G):
                fire_gather(sp, g, g)

            def r_body(r, carry):
                for g in range(_G):
                    j = r * _G + g
                    if sp == 0 and not tail:
                        @pl.when((q > 0) | (r > 0))
                        def _():
                            wait_scatter(sp, g)
                    else:
                        wait_scatter(sp, g)
                    wait_gather(sp, j, g)
                    wvs = [wedge_v[pl.ds(16 * k, 16)] for k in range(4)]
                    lane = [jnp.full((16,), t, jnp.int32) for t in range(16)]

                    @plsc.parallel_loop(0, _CH // 16, unroll=2)
                    def grp_body(jj):
                        ewv = ew_v[sp, pl.ds(j * _CH + 16 * jj, 16)]
                        for t in range(16):
                            e = 16 * jj + t
                            sew = ewv.at[lane[t]].get(
                                mode="promise_in_bounds")
                            for k in range(4):
                                gv = rows_v[g, e, pl.ds(16 * k, 16)]
                                msg_v[g, e, pl.ds(16 * k, 16)] = (
                                    jnp.maximum(gv + sew * wvs[k], 0.0))

                    fire_scatter(sp, j, g)

                    @pl.when(r < _NR - 1)
                    def _():
                        fire_gather(sp, j + _G, g)
                return carry

            lax.fori_loop(0, _NR, r_body, 0)

        def q_body(q, carry):
            super_phase(q, 0)
            super_phase(q, 1)
            return carry

        lax.fori_loop(0, _NSUP // 2, q_body, 0)
        super_phase(_NSUP // 2, 0, tail=True)
        for g in range(_G):
            wait_scatter(0, g)
        plsc.subcore_barrier()
        pltpu.sync_copy(acc_sh.at[pl.ds(s * _RPT, _RPT)],
                        out_hbm.at[c, pl.ds(s * _RPT, _RPT)])

    return sc


_CNCI = 25
_CNSUP = _NCHUNK // _CNCI
_CG = 5


def _make_sc_counts():
    mesh = plsc.VectorSubcoreMesh(core_axis_name="c", subcore_axis_name="s")

    @functools.partial(
        pl.kernel,
        out_type=jax.ShapeDtypeStruct((_NC, _NPAD, 16), jnp.float32),
        mesh=mesh,
        compiler_params=pltpu.CompilerParams(use_tc_tiling_on_sc=False),
        scratch_types=[
            pltpu.VMEM((2, _CNCI, _CH), jnp.int32),
            pltpu.VMEM((_CH, 16), jnp.float32),
            pltpu.VMEM_SHARED((_NPAD, 16), jnp.float32),
        ] + [pltpu.SemaphoreType.DMA] * (_CG + 2),
    )
    def sck(col_hbm, zeros_hbm, out_hbm, col_v, cmsg_v, cnt_sh, *sems):
        ss = sems[:_CG]
        sup = sems[_CG:]
        c = lax.axis_index("c")
        s = lax.axis_index("s")
        wid = s * _NC + c
        pltpu.sync_copy(zeros_hbm.at[pl.ds(s * _RPT, _RPT)],
                        cnt_sh.at[pl.ds(s * _RPT, _RPT)])
        cnt_vec = jnp.maximum(1 - lax.iota(jnp.int32, 16), 0).astype(jnp.float32)

        def fill_body(e, carry):
            cmsg_v[e, :] = cnt_vec
            return carry

        lax.fori_loop(0, _CH, fill_body, 0)
        plsc.subcore_barrier()

        cbase = wid * _NCHUNK

        def load_super(p, slot):
            pltpu.async_copy(col_hbm.at[pl.ds(cbase + p * _CNCI, _CNCI)],
                             col_v.at[slot], sup[slot])

        def wait_super(slot):
            pltpu.make_async_copy(col_hbm.at[pl.ds(cbase, _CNCI)],
                                  col_v.at[slot], sup[slot]).wait()

        def fire_scatter(sp, j, g):
            pltpu.async_copy(cmsg_v, cnt_sh.at[col_v.at[sp, j]], ss[g],
                             add=True)

        def wait_scatter(sp, g):
            pltpu.make_async_copy(cmsg_v, cnt_sh.at[col_v.at[sp, 0]],
                                  ss[g]).wait()

        pltpu.sync_copy(col_hbm.at[pl.ds(cbase, _CNCI)], col_v.at[0])

        def super_phase(q, sp):
            p = 2 * q + sp
            if sp == 0:
                @pl.when(q > 0)
                def _():
                    wait_super(0)
                load_super(p + 1, 1)
            else:
                wait_super(1)

                @pl.when(q < _CNSUP // 2 - 1)
                def _():
                    load_super(p + 1, 0)

            def r_body(r, carry):
                for g in range(_CG):
                    j = r * _CG + g
                    if sp == 0:
                        @pl.when((q > 0) | (r > 0))
                        def _():
                            wait_scatter(sp, g)
                    else:
                        wait_scatter(sp, g)
                    fire_scatter(sp, j, g)
                return carry

            lax.fori_loop(0, _CNCI // _CG, r_body, 0)

        def q_body(q, carry):
            super_phase(q, 0)
            super_phase(q, 1)
            return carry

        lax.fori_loop(0, _CNSUP // 2, q_body, 0)
        for g in range(_CG):
            wait_scatter(1, g)
        plsc.subcore_barrier()
        pltpu.sync_copy(cnt_sh.at[pl.ds(s * _RPT, _RPT)],
                        out_hbm.at[c, pl.ds(s * _RPT, _RPT)])

    return sck


_sc64 = _make_sc()
_sc_cnt = _make_sc_counts()


def kernel(x, u, edge_index, edge_w, loc, W_x, b_x, W1_0, b1_0, W2_0, b2_0,
           W1_1, b1_1, W2_1, b2_1, Wp1, bp1, Wp2, bp2):
    xf = x.reshape(_B * _CITY, _TW * _F)
    ei = edge_index.astype(jnp.int32)
    offs = (jnp.arange(_B, dtype=jnp.int32) * _CITY)[:, None]
    row = (ei[:, 0, :] + offs).reshape(_NE // _CH, _CH)
    col = (ei[:, 1, :] + offs).reshape(_NE // _CH, _CH)
    ew = edge_w.reshape(-1)

    zeros64 = jnp.zeros((_NPAD, 64), jnp.float32)
    zeros16 = jnp.zeros((_NPAD, 16), jnp.float32)

    cnts = _sc_cnt(col, zeros16)[:, :_N]
    cn0, cn1 = cnts[0, :, :8], cnts[1, :, :8]
    x2, g0 = _tc1(xf, W_x, b_x.reshape(1, -1), W1_0[:_XEM], b1_0.reshape(1, -1))
    p0 = _sc64(g0, row, col, ew, W1_0[_XEM], zeros64)[:, :_N]
    nx, g1 = _tc2(x2, p0[0], p0[1], cn0, cn1, W2_0[:_XEM], W2_0[_XEM:],
                  b2_0.reshape(1, -1), W1_1[:_GNNH], b1_1.reshape(1, -1))
    p1 = _sc64(g1, row, col, ew, W1_1[_GNNH], zeros64)[:, :_N]
    res = _tc3(nx, p1[0], p1[1], cn0, cn1,
               W2_1[:_GNNH], W2_1[_GNNH:], b2_1.reshape(1, -1),
               Wp1, bp1.reshape(1, -1), Wp2, bp2.reshape(1, -1))
    return res.reshape(_B, _CITY, _PRED)


# speedup vs baseline: 1.0361x; 1.0361x over previous
import functools

import jax
import jax.numpy as jnp
from jax import lax
from jax.experimental import pallas as pl
from jax.experimental.pallas import tpu as pltpu
from jax.experimental.pallas import tpu_sc as plsc

_B = 2
_CITY = 10000
_E = 320000
_TW = 24
_F = 8
_XEM = 64
_GNNH = 64
_PRED = 12

_N = _B * _CITY
_NE = _B * _E
_NC = 2
_NS = 16
_NW = _NC * _NS
_EPW = _NE // _NW
_CH = 80
_NCHUNK = _EPW // _CH
_NPAD = 20480
_RPT = _NPAD // _NS
_RB = 1000


def _tc1_body(x_ref, wx_ref, bx_ref, w1_ref, b1_ref, x2_ref, g0_ref):
    x2 = jnp.dot(x_ref[...], wx_ref[...], preferred_element_type=jnp.float32)
    x2 = x2 + bx_ref[...]
    x2_ref[...] = x2
    g0 = jnp.dot(x2, w1_ref[...], preferred_element_type=jnp.float32)
    g0_ref[...] = g0 + b1_ref[...]


def _tc1(xf, wx, bx, w1a, b1):
    return pl.pallas_call(
        _tc1_body,
        grid=(_N // _RB,),
        in_specs=[
            pl.BlockSpec((_RB, _TW * _F), lambda i: (i, 0)),
            pl.BlockSpec((_TW * _F, _XEM), lambda i: (0, 0)),
            pl.BlockSpec((1, _XEM), lambda i: (0, 0)),
            pl.BlockSpec((_XEM, _GNNH), lambda i: (0, 0)),
            pl.BlockSpec((1, _GNNH), lambda i: (0, 0)),
        ],
        out_specs=[
            pl.BlockSpec((_RB, _XEM), lambda i: (i, 0)),
            pl.BlockSpec((_RB, _GNNH), lambda i: (i, 0)),
        ],
        out_shape=[
            jax.ShapeDtypeStruct((_N, _XEM), jnp.float32),
            jax.ShapeDtypeStruct((_N, _GNNH), jnp.float32),
        ],
    )(xf, wx, bx, w1a, b1)


def _tc2_body(x2_ref, pa_ref, pb_ref, ca_ref, cb_ref, w2a_ref, w2b_ref, b2_ref,
              w1b_ref, b1b_ref, nx_ref, g1_ref):
    cnt = ca_ref[0][:, :1] + cb_ref[0][:, :1]
    agg = (pa_ref[0] + pb_ref[0]) / jnp.maximum(cnt, 1.0)
    nx = jnp.dot(x2_ref[...], w2a_ref[...], preferred_element_type=jnp.float32)
    nx = nx + jnp.dot(agg, w2b_ref[...], preferred_element_type=jnp.float32)
    nx = jnp.maximum(nx + b2_ref[...], 0.0)
    nx_ref[...] = nx
    g1 = jnp.dot(nx, w1b_ref[...], preferred_element_type=jnp.float32)
    g1_ref[...] = g1 + b1b_ref[...]


def _tc2(x2, p0, cnts, w2a, w2b, b2, w1b, b1b):
    return pl.pallas_call(
        _tc2_body,
        grid=(_N // _RB,),
        in_specs=[
            pl.BlockSpec((_RB, _XEM), lambda i: (i, 0)),
            pl.BlockSpec((1, _RB, _GNNH), lambda i: (0, i, 0)),
            pl.BlockSpec((1, _RB, _GNNH), lambda i: (1, i, 0)),
            pl.BlockSpec((1, _RB, 16), lambda i: (0, i, 0)),
            pl.BlockSpec((1, _RB, 16), lambda i: (1, i, 0)),
            pl.BlockSpec((_XEM, _GNNH), lambda i: (0, 0)),
            pl.BlockSpec((_GNNH, _GNNH), lambda i: (0, 0)),
            pl.BlockSpec((1, _GNNH), lambda i: (0, 0)),
            pl.BlockSpec((_GNNH, _GNNH), lambda i: (0, 0)),
            pl.BlockSpec((1, _GNNH), lambda i: (0, 0)),
        ],
        out_specs=[
            pl.BlockSpec((_RB, _GNNH), lambda i: (i, 0)),
            pl.BlockSpec((_RB, _GNNH), lambda i: (i, 0)),
        ],
        out_shape=[
            jax.ShapeDtypeStruct((_N, _GNNH), jnp.float32),
            jax.ShapeDtypeStruct((_N, _GNNH), jnp.float32),
        ],
    )(x2, p0, p0, cnts, cnts, w2a, w2b, b2, w1b, b1b)


def _tc3_body(nx_ref, pa_ref, pb_ref, ca_ref, cb_ref, w2c_ref, w2d_ref,
              b2_ref, wp1_ref, bp1_ref, wp2_ref, bp2_ref, out_ref):
    cnt = ca_ref[0][:, :1] + cb_ref[0][:, :1]
    agg = (pa_ref[0] + pb_ref[0]) / jnp.maximum(cnt, 1.0)
    nx2 = jnp.dot(nx_ref[...], w2c_ref[...], preferred_element_type=jnp.float32)
    nx2 = nx2 + jnp.dot(agg, w2d_ref[...], preferred_element_type=jnp.float32)
    nx2 = jnp.maximum(nx2 + b2_ref[...], 0.0)
    h = jnp.dot(nx2, wp1_ref[...], preferred_element_type=jnp.float32)
    h = jnp.maximum(h + bp1_ref[...], 0.0)
    o = jnp.dot(h, wp2_ref[...], preferred_element_type=jnp.float32)
    out_ref[...] = jnp.maximum(o + bp2_ref[...], 0.0)


def _tc3(nx, p1, cnts, w2c, w2d, b2, wp1, bp1, wp2, bp2):
    return pl.pallas_call(
        _tc3_body,
        grid=(_N // _RB,),
        in_specs=[
            pl.BlockSpec((_RB, _GNNH), lambda i: (i, 0)),
            pl.BlockSpec((1, _RB, _GNNH), lambda i: (0, i, 0)),
            pl.BlockSpec((1, _RB, _GNNH), lambda i: (1, i, 0)),
            pl.BlockSpec((1, _RB, 16), lambda i: (0, i, 0)),
            pl.BlockSpec((1, _RB, 16), lambda i: (1, i, 0)),
            pl.BlockSpec((_GNNH, _GNNH), lambda i: (0, 0)),
            pl.BlockSpec((_GNNH, _GNNH), lambda i: (0, 0)),
            pl.BlockSpec((1, _GNNH), lambda i: (0, 0)),
            pl.BlockSpec((_GNNH, 16), lambda i: (0, 0)),
            pl.BlockSpec((1, 16), lambda i: (0, 0)),
            pl.BlockSpec((16, _PRED), lambda i: (0, 0)),
            pl.BlockSpec((1, _PRED), lambda i: (0, 0)),
        ],
        out_specs=pl.BlockSpec((_RB, _PRED), lambda i: (i, 0)),
        out_shape=jax.ShapeDtypeStruct((_N, _PRED), jnp.float32),
    )(nx, p1, p1, cnts, cnts, w2c, w2d, b2, wp1, bp1, wp2, bp2)


_G = 2
_NCI = 10
_S = _NCI * _CH
_NSUP = _EPW // _S
_NR = _NCI // _G
_ZB = _RPT // _CH


def _make_sc(with_counts):
    mesh = plsc.VectorSubcoreMesh(core_axis_name="c", subcore_axis_name="s")

    out_type = [jax.ShapeDtypeStruct((_NC, _NPAD, 64), jnp.float32)]
    scratch = [
        pltpu.VMEM((2, _NCI, _CH), jnp.int32),
        pltpu.VMEM((2, _NCI, _CH), jnp.int32),
        pltpu.VMEM((2, _S), jnp.float32),
        pltpu.VMEM((_G, _CH, 64), jnp.float32),
        pltpu.VMEM((_G, _CH, 64), jnp.float32),
        pltpu.VMEM((64,), jnp.float32),
        pltpu.VMEM_SHARED((_NPAD, 64), jnp.float32),
    ]
    nsem = 2 * _G + 2
    if with_counts:
        out_type.append(jax.ShapeDtypeStruct((_NC, _NPAD, 16), jnp.float32))
        scratch.append(pltpu.VMEM((_CH, 16), jnp.float32))
        scratch.append(pltpu.VMEM_SHARED((_NPAD, 16), jnp.float32))
        nsem += _G
    scratch += [pltpu.SemaphoreType.DMA] * nsem

    @functools.partial(
        pl.kernel,
        out_type=tuple(out_type) if with_counts else out_type[0],
        mesh=mesh,
        compiler_params=pltpu.CompilerParams(use_tc_tiling_on_sc=False),
        scratch_types=scratch,
    )
    def sc(g_hbm, row_hbm, col_hbm, ew_hbm, wedge_hbm, *rest):
        if with_counts:
            (out_hbm, cnt_hbm, row_v, col_v, ew_v, rows_v, msg_v, wedge_v,
             acc_sh, cmsg_v, cnt_sh, *sems) = rest
        else:
            (out_hbm, row_v, col_v, ew_v, rows_v, msg_v, wedge_v,
             acc_sh, *sems) = rest
        sg = sems[:_G]
        ss = sems[_G:2 * _G]
        sup = sems[2 * _G:2 * _G + 2]
        if with_counts:
            sk = sems[2 * _G + 2:]
        c = lax.axis_index("c")
        s = lax.axis_index("s")
        wid = s * _NC + c

        @plsc.parallel_loop(0, _CH * 4)
        def _zfill(i):
            msg_v[0, i // 4, pl.ds(16 * (i % 4), 16)] = jnp.zeros(
                (16,), jnp.float32)

        for z in range(_ZB):
            pltpu.sync_copy(msg_v.at[0],
                            acc_sh.at[pl.ds(s * _RPT + z * _CH, _CH)])
        if with_counts:
            @plsc.parallel_loop(0, _CH)
            def _zfill2(i):
                cmsg_v[i, :] = jnp.zeros((16,), jnp.float32)

            for z in range(_ZB):
                pltpu.sync_copy(cmsg_v,
                                cnt_sh.at[pl.ds(s * _RPT + z * _CH, _CH)])
        pltpu.sync_copy(wedge_hbm, wedge_v)
        plsc.subcore_barrier()
        if with_counts:
            onehot = jnp.maximum(1 - lax.iota(jnp.int32, 16), 0).astype(
                jnp.float32)

            @plsc.parallel_loop(0, _CH)
            def _ofill(i):
                cmsg_v[i, :] = onehot

        cbase = wid * _NCHUNK
        ebase = wid * _EPW

        def load_super(p, slot):
            pltpu.async_copy(row_hbm.at[pl.ds(cbase + p * _NCI, _NCI)],
                             row_v.at[slot], sup[slot])
            pltpu.async_copy(col_hbm.at[pl.ds(cbase + p * _NCI, _NCI)],
                             col_v.at[slot], sup[slot])
            pltpu.async_copy(ew_hbm.at[pl.ds(ebase + p * _S, _S)],
                             ew_v.at[slot], sup[slot])

        def wait_super(slot):
            pltpu.make_async_copy(row_hbm.at[pl.ds(cbase, _NCI)],
                                  row_v.at[slot], sup[slot]).wait()
            pltpu.make_async_copy(col_hbm.at[pl.ds(cbase, _NCI)],
                                  col_v.at[slot], sup[slot]).wait()
            pltpu.make_async_copy(ew_hbm.at[pl.ds(ebase, _S)],
                                  ew_v.at[slot], sup[slot]).wait()

        def fire_gather(sp, j, g):
            pltpu.async_copy(g_hbm.at[row_v.at[sp, j]], rows_v.at[g], sg[g])

        def wait_gather(sp, j, g):
            pltpu.make_async_copy(g_hbm.at[row_v.at[sp, j]],
                                  rows_v.at[g], sg[g]).wait()

        def fire_scatter(sp, j, g):
            pltpu.async_copy(msg_v.at[g], acc_sh.at[col_v.at[sp, j]],
                             ss[g], add=True)
            if with_counts:
                pltpu.async_copy(cmsg_v, cnt_sh.at[col_v.at[sp, j]],
                                 sk[g], add=True)

        def wait_scatter(sp, g):
            pltpu.make_async_copy(msg_v.at[g], acc_sh.at[col_v.at[sp, 0]],
                                  ss[g]).wait()
            if with_counts:
                pltpu.make_async_copy(cmsg_v, cnt_sh.at[col_v.at[sp, 0]],
                                      sk[g]).wait()

        pltpu.sync_copy(row_hbm.at[pl.ds(cbase, _NCI)], row_v.at[0])
        pltpu.sync_copy(col_hbm.at[pl.ds(cbase, _NCI)], col_v.at[0])
        pltpu.sync_copy(ew_hbm.at[pl.ds(ebase, _S)], ew_v.at[0])

        def super_phase(q, sp, tail=False):
            p = 2 * q + sp
            if tail:
                wait_super(sp)
            elif sp == 0:
                @pl.when(q > 0)
                def _():
                    wait_super(0)
                load_super(p + 1, 1)
            else:
                wait_super(1)
                load_super(p + 1, 0)
            for g in range(_G):
                fire_gather(sp, g, g)

            def r_body(r, carry):
                for g in range(_G):
                    j = r * _G + g
                    if sp == 0 and not tail:
                        @pl.when((q > 0) | (r > 0))
                        def _():
                            wait_scatter(sp, g)
                    else:
                        wait_scatter(sp, g)
                    wait_gather(sp, j, g)
                    wvs = [wedge_v[pl.ds(16 * k, 16)] for k in range(4)]
                    lane = [jnp.full((16,), t, jnp.int32) for t in range(16)]

                    @plsc.parallel_loop(0, _CH // 16, unroll=2)
                    def grp_body(jj):
                        ewv = ew_v[sp, pl.ds(j * _CH + 16 * jj, 16)]
                        for t in range(16):
                            e = 16 * jj + t
                            sew = ewv.at[lane[t]].get(
                                mode="promise_in_bounds")
                            for k in range(4):
                                gv = rows_v[g, e, pl.ds(16 * k, 16)]
                                msg_v[g, e, pl.ds(16 * k, 16)] = (
                                    jnp.maximum(gv + sew * wvs[k], 0.0))

                    fire_scatter(sp, j, g)

                    @pl.when(r < _NR - 1)
                    def _():
                        fire_gather(sp, j + _G, g)
                return carry

            lax.fori_loop(0, _NR, r_body, 0)

        def q_body(q, carry):
            super_phase(q, 0)
            super_phase(q, 1)
            return carry

        lax.fori_loop(0, _NSUP // 2, q_body, 0)
        super_phase(_NSUP // 2, 0, tail=True)
        for g in range(_G):
            wait_scatter(0, g)
        plsc.subcore_barrier()
        pltpu.sync_copy(acc_sh.at[pl.ds(s * _RPT, _RPT)],
                        out_hbm.at[c, pl.ds(s * _RPT, _RPT)])
        if with_counts:
            pltpu.sync_copy(cnt_sh.at[pl.ds(s * _RPT, _RPT)],
                            cnt_hbm.at[c, pl.ds(s * _RPT, _RPT)])

    return sc


_sc_main0 = _make_sc(True)
_sc_main1 = _make_sc(False)


def kernel(x, u, edge_index, edge_w, loc, W_x, b_x, W1_0, b1_0, W2_0, b2_0,
           W1_1, b1_1, W2_1, b2_1, Wp1, bp1, Wp2, bp2):
    xf = x.reshape(_B * _CITY, _TW * _F)
    ei = edge_index.astype(jnp.int32)
    offs = (jnp.arange(_B, dtype=jnp.int32) * _CITY)[:, None]
    row = (ei[:, 0, :] + offs).reshape(_NE // _CH, _CH)
    col = (ei[:, 1, :] + offs).reshape(_NE // _CH, _CH)
    ew = edge_w.reshape(-1)

    x2, g0 = _tc1(xf, W_x, b_x.reshape(1, -1), W1_0[:_XEM], b1_0.reshape(1, -1))
    p0, cnts = _sc_main0(g0, row, col, ew, W1_0[_XEM])
    nx, g1 = _tc2(x2, p0, cnts, W2_0[:_XEM], W2_0[_XEM:],
                  b2_0.reshape(1, -1), W1_1[:_GNNH], b1_1.reshape(1, -1))
    p1 = _sc_main1(g1, row, col, ew, W1_1[_GNNH])
    res = _tc3(nx, p1, cnts,
               W2_1[:_GNNH], W2_1[_GNNH:], b2_1.reshape(1, -1),
               Wp1, bp1.reshape(1, -1), Wp2, bp2.reshape(1, -1))
    return res.reshape(_B, _CITY, _PRED)


# speedup vs baseline: 1.1498x; 1.1097x over previous
import functools

import jax
import jax.numpy as jnp
from jax import lax
from jax.experimental import pallas as pl
from jax.experimental.pallas import tpu as pltpu
from jax.experimental.pallas import tpu_sc as plsc

_B = 2
_CITY = 10000
_E = 320000
_TW = 24
_F = 8
_XEM = 64
_GNNH = 64
_PRED = 12

_N = _B * _CITY
_NE = _B * _E
_NC = 2
_NS = 16
_NW = _NC * _NS
_EPW = _NE // _NW
_CH = 80
_NCHUNK = _EPW // _CH
_NPAD = 20480
_RPT = _NPAD // _NS
_RB = 4000


def _tc1_body(x_ref, wx_ref, bx_ref, w1_ref, b1_ref, x2_ref, g0_ref):
    x2 = jnp.dot(x_ref[...], wx_ref[...], preferred_element_type=jnp.float32)
    x2 = x2 + bx_ref[...][None, :]
    x2_ref[...] = x2
    g0 = jnp.dot(x2, w1_ref[...][:_XEM], preferred_element_type=jnp.float32)
    g0_ref[...] = g0 + b1_ref[...][None, :]


def _tc1(xf, wx, bx, w1a, b1):
    return pl.pallas_call(
        _tc1_body,
        grid=(_N // _RB,),
        in_specs=[
            pl.BlockSpec((_RB, _TW * _F), lambda i: (i, 0)),
            pl.BlockSpec((_TW * _F, _XEM), lambda i: (0, 0)),
            pl.BlockSpec((_XEM,), lambda i: (0,)),
            pl.BlockSpec((_XEM + 1, _GNNH), lambda i: (0, 0)),
            pl.BlockSpec((_GNNH,), lambda i: (0,)),
        ],
        out_specs=[
            pl.BlockSpec((_RB, _XEM), lambda i: (i, 0)),
            pl.BlockSpec((_RB, _GNNH), lambda i: (i, 0)),
        ],
        out_shape=[
            jax.ShapeDtypeStruct((_N, _XEM), jnp.float32),
            jax.ShapeDtypeStruct((_N, _GNNH), jnp.float32),
        ],
    )(xf, wx, bx, w1a, b1)


def _tc2_body(x2_ref, pa_ref, pb_ref, ca_ref, cb_ref, w2_ref, b2_ref,
              w1b_ref, b1b_ref, nx_ref, g1_ref):
    cnt = ca_ref[0][:, :1] + cb_ref[0][:, :1]
    agg = (pa_ref[0] + pb_ref[0]) / jnp.maximum(cnt, 1.0)
    w2 = w2_ref[...]
    nx = jnp.dot(x2_ref[...], w2[:_XEM], preferred_element_type=jnp.float32)
    nx = nx + jnp.dot(agg, w2[_XEM:], preferred_element_type=jnp.float32)
    nx = jnp.maximum(nx + b2_ref[...][None, :], 0.0)
    nx_ref[...] = nx
    g1 = jnp.dot(nx, w1b_ref[...][:_GNNH], preferred_element_type=jnp.float32)
    g1_ref[...] = g1 + b1b_ref[...][None, :]


def _tc2(x2, p0, cnts, w2, b2, w1b, b1b):
    return pl.pallas_call(
        _tc2_body,
        grid=(_N // _RB,),
        in_specs=[
            pl.BlockSpec((_RB, _XEM), lambda i: (i, 0)),
            pl.BlockSpec((1, _RB, _GNNH), lambda i: (0, i, 0)),
            pl.BlockSpec((1, _RB, _GNNH), lambda i: (1, i, 0)),
            pl.BlockSpec((1, _RB, 16), lambda i: (0, i, 0)),
            pl.BlockSpec((1, _RB, 16), lambda i: (1, i, 0)),
            pl.BlockSpec((_XEM + _GNNH, _GNNH), lambda i: (0, 0)),
            pl.BlockSpec((_GNNH,), lambda i: (0,)),
            pl.BlockSpec((_GNNH + 1, _GNNH), lambda i: (0, 0)),
            pl.BlockSpec((_GNNH,), lambda i: (0,)),
        ],
        out_specs=[
            pl.BlockSpec((_RB, _GNNH), lambda i: (i, 0)),
            pl.BlockSpec((_RB, _GNNH), lambda i: (i, 0)),
        ],
        out_shape=[
            jax.ShapeDtypeStruct((_N, _GNNH), jnp.float32),
            jax.ShapeDtypeStruct((_N, _GNNH), jnp.float32),
        ],
    )(x2, p0, p0, cnts, cnts, w2, b2, w1b, b1b)


def _tc3_body(nx_ref, pa_ref, pb_ref, ca_ref, cb_ref, w2_ref,
              b2_ref, wp1_ref, bp1_ref, wp2_ref, bp2_ref, out_ref):
    cnt = ca_ref[0][:, :1] + cb_ref[0][:, :1]
    agg = (pa_ref[0] + pb_ref[0]) / jnp.maximum(cnt, 1.0)
    w2 = w2_ref[...]
    nx2 = jnp.dot(nx_ref[...], w2[:_GNNH], preferred_element_type=jnp.float32)
    nx2 = nx2 + jnp.dot(agg, w2[_GNNH:], preferred_element_type=jnp.float32)
    nx2 = jnp.maximum(nx2 + b2_ref[...][None, :], 0.0)
    h = jnp.dot(nx2, wp1_ref[...], preferred_element_type=jnp.float32)
    h = jnp.maximum(h + bp1_ref[...][None, :], 0.0)
    o = jnp.dot(h, wp2_ref[...], preferred_element_type=jnp.float32)
    out_ref[...] = jnp.maximum(o + bp2_ref[...][None, :], 0.0)


def _tc3(nx, p1, cnts, w2, b2, wp1, bp1, wp2, bp2):
    return pl.pallas_call(
        _tc3_body,
        grid=(_N // _RB,),
        in_specs=[
            pl.BlockSpec((_RB, _GNNH), lambda i: (i, 0)),
            pl.BlockSpec((1, _RB, _GNNH), lambda i: (0, i, 0)),
            pl.BlockSpec((1, _RB, _GNNH), lambda i: (1, i, 0)),
            pl.BlockSpec((1, _RB, 16), lambda i: (0, i, 0)),
            pl.BlockSpec((1, _RB, 16), lambda i: (1, i, 0)),
            pl.BlockSpec((2 * _GNNH, _GNNH), lambda i: (0, 0)),
            pl.BlockSpec((_GNNH,), lambda i: (0,)),
            pl.BlockSpec((_GNNH, 16), lambda i: (0, 0)),
            pl.BlockSpec((16,), lambda i: (0,)),
            pl.BlockSpec((16, _PRED), lambda i: (0, 0)),
            pl.BlockSpec((_PRED,), lambda i: (0,)),
        ],
        out_specs=pl.BlockSpec((_RB, _PRED), lambda i: (i, 0)),
        out_shape=jax.ShapeDtypeStruct((_N, _PRED), jnp.float32),
    )(nx, p1, p1, cnts, cnts, w2, b2, wp1, bp1, wp2, bp2)


_G = 2
_NCI = 10
_S = _NCI * _CH
_NSUP = _EPW // _S
_NR = _NCI // _G
_ZB = _RPT // _CH


def _make_sc(with_counts):
    mesh = plsc.VectorSubcoreMesh(core_axis_name="c", subcore_axis_name="s")

    out_type = [jax.ShapeDtypeStruct((_NC, _NPAD, 64), jnp.float32)]
    scratch = [
        pltpu.VMEM((2, _NCI, _CH), jnp.int32),
        pltpu.VMEM((2, _NCI, _CH), jnp.int32),
        pltpu.VMEM((2, _S), jnp.float32),
        pltpu.VMEM((_G, _CH, 64), jnp.float32),
        pltpu.VMEM((_G, _CH, 64), jnp.float32),
        pltpu.VMEM((64,), jnp.float32),
        pltpu.VMEM_SHARED((_NPAD, 64), jnp.float32),
    ]
    nsem = 2 * _G + 2
    if with_counts:
        out_type.append(jax.ShapeDtypeStruct((_NC, _NPAD, 16), jnp.float32))
        scratch.append(pltpu.VMEM((_CH, 16), jnp.float32))
        scratch.append(pltpu.VMEM_SHARED((_NPAD, 16), jnp.float32))
        nsem += _G
    scratch += [pltpu.SemaphoreType.DMA] * nsem

    @functools.partial(
        pl.kernel,
        out_type=tuple(out_type) if with_counts else out_type[0],
        mesh=mesh,
        compiler_params=pltpu.CompilerParams(use_tc_tiling_on_sc=False),
        scratch_types=scratch,
    )
    def sc(g_hbm, ei_hbm, ew_hbm, w1_hbm, *rest):
        if with_counts:
            (out_hbm, cnt_hbm, row_v, col_v, ew_v, rows_v, msg_v, wedge_v,
             acc_sh, cmsg_v, cnt_sh, *sems) = rest
        else:
            (out_hbm, row_v, col_v, ew_v, rows_v, msg_v, wedge_v,
             acc_sh, *sems) = rest
        sg = sems[:_G]
        ss = sems[_G:2 * _G]
        sup = sems[2 * _G:2 * _G + 2]
        if with_counts:
            sk = sems[2 * _G + 2:]
        c = lax.axis_index("c")
        s = lax.axis_index("s")
        wid = s * _NC + c
        b = (wid >= _NS).astype(jnp.int32)

        @plsc.parallel_loop(0, _CH * 4)
        def _zfill(i):
            msg_v[0, i // 4, pl.ds(16 * (i % 4), 16)] = jnp.zeros(
                (16,), jnp.float32)

        for z in range(_ZB):
            pltpu.sync_copy(msg_v.at[0],
                            acc_sh.at[pl.ds(s * _RPT + z * _CH, _CH)])
        if with_counts:
            @plsc.parallel_loop(0, _CH)
            def _zfill2(i):
                cmsg_v[i, :] = jnp.zeros((16,), jnp.float32)

            for z in range(_ZB):
                pltpu.sync_copy(cmsg_v,
                                cnt_sh.at[pl.ds(s * _RPT + z * _CH, _CH)])
        pltpu.sync_copy(w1_hbm.at[64], wedge_v)
        plsc.subcore_barrier()
        if with_counts:
            onehot = jnp.maximum(1 - lax.iota(jnp.int32, 16), 0).astype(
                jnp.float32)

            @plsc.parallel_loop(0, _CH)
            def _ofill(i):
                cmsg_v[i, :] = onehot

        cbase = wid * _NCHUNK - b * (_E // _CH)
        ebase = wid * _EPW - b * _E
        g_b = g_hbm.at[pl.ds(b * _CITY, _CITY)]
        acc_b = acc_sh.at[pl.ds(b * _CITY, _CITY)]
        if with_counts:
            cnt_b = cnt_sh.at[pl.ds(b * _CITY, _CITY)]

        def load_super(p, slot):
            pltpu.async_copy(ei_hbm.at[b, 0, pl.ds(cbase + p * _NCI, _NCI)],
                             row_v.at[slot], sup[slot])
            pltpu.async_copy(ei_hbm.at[b, 1, pl.ds(cbase + p * _NCI, _NCI)],
                             col_v.at[slot], sup[slot])
            pltpu.async_copy(ew_hbm.at[b, pl.ds(ebase + p * _S, _S)],
                             ew_v.at[slot], sup[slot])

        def wait_super(slot):
            pltpu.make_async_copy(ei_hbm.at[b, 0, pl.ds(cbase, _NCI)],
                                  row_v.at[slot], sup[slot]).wait()
            pltpu.make_async_copy(ei_hbm.at[b, 1, pl.ds(cbase, _NCI)],
                                  col_v.at[slot], sup[slot]).wait()
            pltpu.make_async_copy(ew_hbm.at[b, pl.ds(ebase, _S)],
                                  ew_v.at[slot], sup[slot]).wait()

        def fire_gather(sp, j, g):
            pltpu.async_copy(g_b.at[row_v.at[sp, j]], rows_v.at[g], sg[g])

        def wait_gather(sp, j, g):
            pltpu.make_async_copy(g_b.at[row_v.at[sp, j]],
                                  rows_v.at[g], sg[g]).wait()

        def fire_scatter(sp, j, g):
            pltpu.async_copy(msg_v.at[g], acc_b.at[col_v.at[sp, j]],
                             ss[g], add=True)
            if with_counts:
                pltpu.async_copy(cmsg_v, cnt_b.at[col_v.at[sp, j]],
                                 sk[g], add=True)

        def wait_scatter(sp, g):
            pltpu.make_async_copy(msg_v.at[g], acc_b.at[col_v.at[sp, 0]],
                                  ss[g]).wait()
            if with_counts:
                pltpu.make_async_copy(cmsg_v, cnt_b.at[col_v.at[sp, 0]],
                                      sk[g]).wait()

        pltpu.sync_copy(ei_hbm.at[b, 0, pl.ds(cbase, _NCI)], row_v.at[0])
        pltpu.sync_copy(ei_hbm.at[b, 1, pl.ds(cbase, _NCI)], col_v.at[0])
        pltpu.sync_copy(ew_hbm.at[b, pl.ds(ebase, _S)], ew_v.at[0])

        def super_phase(q, sp, tail=False):
            p = 2 * q + sp
            if tail:
                wait_super(sp)
            elif sp == 0:
                @pl.when(q > 0)
                def _():
                    wait_super(0)
                load_super(p + 1, 1)
            else:
                wait_super(1)
                load_super(p + 1, 0)
            for g in range(_G):
                fire_gather(sp, g, g)

            def r_body(r, carry):
                for g in range(_G):
                    j = r * _G + g
                    if sp == 0 and not tail:
                        @pl.when((q > 0) | (r > 0))
                        def _():
                            wait_scatter(sp, g)
                    else:
                        wait_scatter(sp, g)
                    wait_gather(sp, j, g)
                    wvs = [wedge_v[pl.ds(16 * k, 16)] for k in range(4)]
                    lane = [jnp.full((16,), t, jnp.int32) for t in range(16)]

                    @plsc.parallel_loop(0, _CH // 16, unroll=2)
                    def grp_body(jj):
                        ewv = ew_v[sp, pl.ds(j * _CH + 16 * jj, 16)]
                        for t in range(16):
                            e = 16 * jj + t
                            sew = ewv.at[lane[t]].get(
                                mode="promise_in_bounds")
                            for k in range(4):
                                gv = rows_v[g, e, pl.ds(16 * k, 16)]
                                msg_v[g, e, pl.ds(16 * k, 16)] = (
                                    jnp.maximum(gv + sew * wvs[k], 0.0))

                    fire_scatter(sp, j, g)

                    @pl.when(r < _NR - 1)
                    def _():
                        fire_gather(sp, j + _G, g)
                return carry

            lax.fori_loop(0, _NR, r_body, 0)

        def q_body(q, carry):
            super_phase(q, 0)
            super_phase(q, 1)
            return carry

        lax.fori_loop(0, _NSUP // 2, q_body, 0)
        super_phase(_NSUP // 2, 0, tail=True)
        for g in range(_G):
            wait_scatter(0, g)
        plsc.subcore_barrier()
        pltpu.sync_copy(acc_sh.at[pl.ds(s * _RPT, _RPT)],
                        out_hbm.at[c, pl.ds(s * _RPT, _RPT)])
        if with_counts:
            pltpu.sync_copy(cnt_sh.at[pl.ds(s * _RPT, _RPT)],
                            cnt_hbm.at[c, pl.ds(s * _RPT, _RPT)])

    return sc


_sc_main0 = _make_sc(True)
_sc_main1 = _make_sc(False)


def kernel(x, u, edge_index, edge_w, loc, W_x, b_x, W1_0, b1_0, W2_0, b2_0,
           W1_1, b1_1, W2_1, b2_1, Wp1, bp1, Wp2, bp2):
    xf = x.reshape(_B * _CITY, _TW * _F)
    ei = edge_index.astype(jnp.int32).reshape(_B, 2, _E // _CH, _CH)

    x2, g0 = _tc1(xf, W_x, b_x, W1_0, b1_0)
    p0, cnts = _sc_main0(g0, ei, edge_w, W1_0)
    nx, g1 = _tc2(x2, p0, cnts, W2_0, b2_0, W1_1, b1_1)
    p1 = _sc_main1(g1, ei, edge_w, W1_1)
    res = _tc3(nx, p1, cnts, W2_1, b2_1, Wp1, bp1, Wp2, bp2)
    return res.reshape(_B, _CITY, _PRED)
